# Initial kernel scaffold; baseline (speedup 1.0000x reference)
import jax
import jax.numpy as jnp
from jax.experimental import pallas as pl


def kernel(x, edge_index, edge_attr, mask_node_indices, prelu_a, W_enc, emb1, emb2, mlp_w1, mlp_b1, mlp_w2, mlp_b2, out_w, out_b):
    raise NotImplementedError("write your pallas kernel here")


# speedup vs baseline: 1.9912x; 1.9912x over previous
import functools

import jax
import jax.numpy as jnp
from jax import lax
from jax.experimental import pallas as pl
from jax.experimental.pallas import tpu as pltpu

N_NODES = 10000
HID = 256
OUT = 128
ROW_BLK = 1000
N_BLKS = N_NODES // ROW_BLK
MASK_PAD = 1536


def _encode_body(x_ref, a_ref, w_ref, mask_ref, h_ref):
    i = pl.program_id(0)
    xb = x_ref[...]
    a = a_ref[0, 0]
    act = jnp.where(xb >= 0, xb, a * xb)
    rows = lax.broadcasted_iota(jnp.int32, (ROW_BLK, 128), 0) + i * ROW_BLK
    hit = jnp.zeros((ROW_BLK, 128), dtype=jnp.bool_)
    for j in range(MASK_PAD // 128):
        hit = jnp.logical_or(hit, rows == mask_ref[j, :][None, :])
    anyhit = jnp.any(hit, axis=1, keepdims=True)
    act = jnp.where(anyhit, 0.0, act)
    h_ref[...] = lax.dot_general(act, w_ref[...], (((1,), (1,)), ((), ())),
                                 preferred_element_type=jnp.float32)


def _encode(x, prelu_a, W_enc, mask_pad):
    return pl.pallas_call(
        _encode_body,
        grid=(N_BLKS,),
        in_specs=[
            pl.BlockSpec((ROW_BLK, HID), lambda i: (i, 0)),
            pl.BlockSpec((1, 1), lambda i: (0, 0)),
            pl.BlockSpec((HID, HID), lambda i: (0, 0)),
            pl.BlockSpec((MASK_PAD // 128, 128), lambda i: (0, 0)),
        ],
        out_specs=pl.BlockSpec((ROW_BLK, HID), lambda i: (i, 0)),
        out_shape=jax.ShapeDtypeStruct((N_NODES, HID), jnp.float32),
    )(x, prelu_a, W_enc, mask_pad)


def _decode_body(slo_ref, shi_ref, h_ref, h0_ref, h1_ref, emb_ref,
                 w1_ref, b1_ref, w2_ref, b2_ref, ow_ref, ob_ref, out_ref):
    hist = h0_ref[...] + h1_ref[...]
    hist = hist + (lax.broadcasted_iota(jnp.int32, (ROW_BLK, 16), 1) == 9)
    cnt = jnp.sum(hist, axis=1, keepdims=True)
    e = lax.dot_general(hist, emb_ref[...], (((1,), (0,)), ((), ())),
                        preferred_element_type=jnp.float32)
    s = jnp.concatenate([slo_ref[...], shi_ref[...]], axis=1)
    agg = (s + h_ref[...] + e) / jnp.maximum(cnt, 1.0)
    g = lax.dot_general(agg, w1_ref[...], (((1,), (1,)), ((), ())),
                        preferred_element_type=jnp.float32)
    g = jnp.maximum(g + b1_ref[...], 0.0)
    g = lax.dot_general(g, w2_ref[...], (((1,), (1,)), ((), ())),
                        preferred_element_type=jnp.float32) + b2_ref[...]
    out_ref[...] = lax.dot_general(g, ow_ref[...], (((1,), (1,)), ((), ())),
                                   preferred_element_type=jnp.float32) + ob_ref[...]


def _decode(s_lo, s_hi, h, hist0, hist1, emb12p,
            mlp_w1, mlp_b1, mlp_w2, mlp_b2, out_w, out_b):
    full = lambda shape: pl.BlockSpec(shape, lambda i: tuple(0 for _ in shape))
    blk = lambda c: pl.BlockSpec((ROW_BLK, c), lambda i: (i, 0))
    return pl.pallas_call(
        _decode_body,
        grid=(N_BLKS,),
        in_specs=[
            blk(128), blk(128), blk(HID), blk(16), blk(16),
            full((16, HID)),
            full((2 * HID, HID)), full((1, 2 * HID)),
            full((HID, 2 * HID)), full((1, HID)),
            full((OUT, HID)), full((1, OUT)),
        ],
        out_specs=blk(OUT),
        out_shape=jax.ShapeDtypeStruct((N_NODES, OUT), jnp.float32),
    )(s_lo, s_hi, h, hist0, hist1, emb12p,
      mlp_w1, mlp_b1.reshape(1, -1), mlp_w2, mlp_b2.reshape(1, -1),
      out_w, out_b.reshape(1, -1))


def kernel(x, edge_index, edge_attr, mask_node_indices, prelu_a, W_enc, emb1,
           emb2, mlp_w1, mlp_b1, mlp_w2, mlp_b2, out_w, out_b):
    mask_pad = jnp.concatenate(
        [mask_node_indices,
         jnp.full((MASK_PAD - mask_node_indices.shape[0],), -1, jnp.int32)]
    ).reshape(MASK_PAD // 128, 128)
    idx0 = jnp.array([0, 0, 0, 1, 1, 1, 2, 2, 2, 4, 0, 0, 0, 0, 0, 0], jnp.int32)
    idx1 = jnp.array([0, 1, 2, 0, 1, 2, 0, 1, 2, 0, 0, 0, 0, 0, 0, 0], jnp.int32)
    emb12p = emb1[idx0] + emb2[idx1]
    emb12p = emb12p * (jnp.arange(16, dtype=jnp.int32) <= 9)[:, None]

    h = _encode(x, prelu_a.reshape(1, 1), W_enc, mask_pad)

    src, dst = edge_index[0], edge_index[1]
    combo = edge_attr[:, 0] * 3 + edge_attr[:, 1]
    ssum = jax.ops.segment_sum(h[src], dst, num_segments=N_NODES)
    onehot = (combo[:, None] == jnp.arange(16, dtype=jnp.int32)[None, :]
              ).astype(jnp.float32)
    hist = jax.ops.segment_sum(onehot, dst, num_segments=N_NODES)

    return _decode(ssum[:, :128], ssum[:, 128:], h, hist,
                   jnp.zeros_like(hist), emb12p,
                   mlp_w1, mlp_b1, mlp_w2, mlp_b2, out_w, out_b)
